# baseline (speedup 1.0000x reference)
import functools

import jax
import jax.numpy as jnp
from jax import lax
from jax.experimental import pallas as pl
from jax.experimental.pallas import tpu as pltpu
from jax.experimental.pallas import tpu_sc as plsc

N_ROWS = 2_000_000
L = 16
NC, NS = 2, 16
NW = NC * NS
VECS = N_ROWS // L
CHUNK_VECS = 625
CHUNK_ROWS = CHUNK_VECS * L
N_CHUNKS = VECS // CHUNK_VECS
BASE_CHUNKS = N_CHUNKS // NW
EXTRA = N_CHUNKS - BASE_CHUNKS * NW

_mesh = plsc.VectorSubcoreMesh(core_axis_name="c", subcore_axis_name="s")


@functools.partial(
    pl.kernel,
    mesh=_mesh,
    compiler_params=pltpu.CompilerParams(needs_layout_passes=False),
    out_type=jax.ShapeDtypeStruct((NW, L), jnp.float32),
    scratch_types=[
        pltpu.VMEM((2 * CHUNK_ROWS,), jnp.float32),
        pltpu.VMEM((CHUNK_ROWS,), jnp.int32),
        pltpu.VMEM((L,), jnp.float32),
    ],
)
def _count_matches(logits_hbm, tgt_hbm, partials_hbm, buf_o, buf_t, acc_v):
    cid = lax.axis_index("c")
    sid = lax.axis_index("s")
    wid = sid * NC + cid
    first = wid * BASE_CHUNKS + jnp.minimum(wid, EXTRA)
    nch = BASE_CHUNKS + jnp.where(wid < EXTRA, 1, 0)

    even_base = lax.iota(jnp.int32, L) * 2

    def chunk_body(ci, acc):
        c = first + ci
        pltpu.sync_copy(
            logits_hbm.at[pl.ds(c * (2 * CHUNK_ROWS), 2 * CHUNK_ROWS)], buf_o)
        pltpu.sync_copy(tgt_hbm.at[pl.ds(c * CHUNK_ROWS, CHUNK_ROWS)], buf_t)

        def vec_body(j, acc):
            idx = even_base + j * (2 * L)
            o0 = plsc.load_gather(buf_o, [idx])
            o1 = plsc.load_gather(buf_o, [idx + 1])
            t = buf_t[pl.ds(j * L, L)]
            m = (o1 > o0) == (t == 1)
            return acc + jnp.where(m, 1.0, 0.0).astype(jnp.float32)

        return lax.fori_loop(0, CHUNK_VECS, vec_body, acc)

    acc = lax.fori_loop(0, nch, chunk_body, jnp.zeros((L,), jnp.float32))
    acc_v[...] = acc
    pltpu.sync_copy(acc_v, partials_hbm.at[wid])


def kernel(output, target, segments):
    del segments
    partials = _count_matches(output.reshape(-1), target)
    return jnp.sum(partials) / jnp.float32(N_ROWS)
